# baseline (speedup 1.0000x reference)
import dataclasses
import functools

import jax
import jax.numpy as jnp
from jax import lax
from jax.experimental import pallas as pl
from jax.experimental.pallas import tpu as pltpu
from jax.experimental.pallas import tpu_sc as plsc

N = 10000
D = 128
E = 320000

NC = 2
NS = 16
NW = NC * NS
CH = 128
CPW = 80
EP = NW * CPW * CH
NP = 10240
RPS = NP // NS

_mesh = plsc.VectorSubcoreMesh(core_axis_name="c", subcore_axis_name="s")

_cp = pltpu.CompilerParams()
if "needs_layout_passes" in pltpu.CompilerParams.__dataclass_fields__:
  _cp = dataclasses.replace(_cp, needs_layout_passes=False)


@functools.partial(
    pl.kernel,
    out_type=jax.ShapeDtypeStruct((NC, NP), jnp.float32),
    mesh=_mesh,
    scratch_types=[
        pltpu.VMEM((CPW, CH), jnp.int32),
        pltpu.VMEM((CPW, CH), jnp.float32),
        pltpu.VMEM((RPS,), jnp.float32),
        pltpu.VMEM_SHARED((NP,), jnp.float32),
    ],
)
def _deg_kernel(col_hbm, ew_hbm, out_hbm, colv, ewv, zb, acc):
  c = lax.axis_index("c")
  s = lax.axis_index("s")
  w = s * NC + c

  z = jnp.zeros((16,), jnp.float32)

  @pl.loop(0, RPS // 16)
  def _(i):
    zb[pl.ds(i * 16, 16)] = z

  pltpu.sync_copy(zb, acc.at[pl.ds(s * RPS, RPS)])
  plsc.subcore_barrier()

  pltpu.sync_copy(col_hbm.at[w], colv)
  pltpu.sync_copy(ew_hbm.at[w], ewv)

  @pl.loop(0, CPW)
  def _(j):
    pltpu.sync_copy(ewv.at[j], acc.at[colv.at[j]], add=True)

  plsc.subcore_barrier()
  pltpu.sync_copy(acc.at[pl.ds(s * RPS, RPS)], out_hbm.at[c, pl.ds(s * RPS, RPS)])


_PH = 2
_CPP = CPW // _PH
_NSH = 10112
_RSC = _NSH // NS


@functools.partial(
    pl.kernel,
    out_type=jax.ShapeDtypeStruct((NC, NP, D), jnp.float32),
    mesh=_mesh,
    scratch_types=[
        pltpu.VMEM((_CPP, CH), jnp.int32),
        pltpu.VMEM((_CPP, CH), jnp.int32),
        pltpu.VMEM((_CPP, CH), jnp.float32),
        pltpu.VMEM((CH // 2, D), jnp.float32),
        pltpu.VMEM((CH // 2, D), jnp.float32),
        pltpu.VMEM((CH, D), jnp.float32),
        pltpu.VMEM_SHARED((_NSH, D), jnp.float32),
        pltpu.SemaphoreType.DMA,
        pltpu.SemaphoreType.DMA,
    ],
    compiler_params=_cp,
)
def _agg_kernel(hp_hbm, row_hbm, col_hbm, ew_hbm, out_hbm,
                rowv, colv, ewv, gA, gB, msgb, acc, semA, semB):
  c = lax.axis_index("c")
  s = lax.axis_index("s")
  w = s * NC + c

  z = jnp.zeros((16,), jnp.float32)

  @pl.loop(0, CH)
  def _(r):
    for v in range(8):
      msgb[r, pl.ds(v * 16, 16)] = z

  for t in range(4):
    pltpu.sync_copy(msgb, acc.at[pl.ds(s * _RSC + t * CH, CH)])
  pltpu.sync_copy(msgb.at[pl.ds(0, _RSC - 4 * CH)],
                  acc.at[pl.ds(s * _RSC + 4 * CH, _RSC - 4 * CH)])
  plsc.subcore_barrier()

  HC = CH // 2

  def gstart(j, half, buf, sem):
    pltpu.make_async_copy(hp_hbm.at[rowv.at[j, pl.ds(half * HC, HC)]],
                          buf, sem).start()

  def gwait(j, half, buf, sem):
    pltpu.make_async_copy(hp_hbm.at[rowv.at[j, pl.ds(half * HC, HC)]],
                          buf, sem).wait()

  def scale(buf, j, half):
    @pl.loop(0, HC // 16)
    def _(k):
      for e in range(16):
        idx = k * 16 + e
        eidx = half * HC + idx
        spl = plsc.load_gather(
            ewv,
            [jnp.full((16,), j, jnp.int32), jnp.full((16,), eidx, jnp.int32)],
        )
        for v in range(8):
          msgb[eidx, pl.ds(v * 16, 16)] = buf[idx, pl.ds(v * 16, 16)] * spl

  for p in range(_PH):
    pltpu.sync_copy(row_hbm.at[w, pl.ds(p * _CPP, _CPP)], rowv)
    pltpu.sync_copy(col_hbm.at[w, pl.ds(p * _CPP, _CPP)], colv)
    pltpu.sync_copy(ew_hbm.at[w, pl.ds(p * _CPP, _CPP)], ewv)

    gstart(0, 0, gA, semA)
    gstart(0, 1, gB, semB)

    @pl.loop(0, _CPP)
    def _(j):
      gwait(j, 0, gA, semA)
      scale(gA, j, 0)

      @pl.when(j + 1 < _CPP)
      def _():
        gstart(j + 1, 0, gA, semA)

      gwait(j, 1, gB, semB)
      scale(gB, j, 1)

      @pl.when(j + 1 < _CPP)
      def _():
        gstart(j + 1, 1, gB, semB)

      pltpu.sync_copy(msgb, acc.at[colv.at[j]], add=True)

  plsc.subcore_barrier()
  pltpu.sync_copy(acc.at[pl.ds(s * _RSC, _RSC)],
                  out_hbm.at[c, pl.ds(s * _RSC, _RSC)])


@functools.partial(
    pl.kernel,
    out_type=jax.ShapeDtypeStruct((NC, NP), jnp.float32),
    mesh=_mesh,
    scratch_types=[
        pltpu.VMEM((CPW, CH), jnp.int32),
        pltpu.VMEM((CPW, CH), jnp.int32),
        pltpu.VMEM((CPW, CH), jnp.float32),
        pltpu.VMEM((1, CH), jnp.float32),
        pltpu.VMEM((NP,), jnp.float32),
        pltpu.VMEM((RPS,), jnp.float32),
        pltpu.VMEM_SHARED((NP,), jnp.float32),
    ],
    compiler_params=_cp,
)
def _aggs_kernel(h_hbm, row_hbm, col_hbm, ew_hbm, out_hbm,
                 rowv, colv, ewv, msg, tbl, zb, acc):
  c = lax.axis_index("c")
  s = lax.axis_index("s")
  w = s * NC + c

  z = jnp.zeros((16,), jnp.float32)

  @pl.loop(0, RPS // 16)
  def _(i):
    zb[pl.ds(i * 16, 16)] = z

  pltpu.sync_copy(zb, acc.at[pl.ds(s * RPS, RPS)])
  plsc.subcore_barrier()

  pltpu.sync_copy(h_hbm, tbl)
  pltpu.sync_copy(row_hbm.at[w], rowv)
  pltpu.sync_copy(col_hbm.at[w], colv)
  pltpu.sync_copy(ew_hbm.at[w], ewv)

  @pl.loop(0, CPW)
  def _(j):
    @pl.loop(0, CH // 16)
    def _(k):
      row16 = rowv[j, pl.ds(k * 16, 16)]
      ew16 = ewv[j, pl.ds(k * 16, 16)]
      vals = plsc.load_gather(tbl, [row16]) * ew16
      msg[0, pl.ds(k * 16, 16)] = vals

    pltpu.sync_copy(msg.at[0], acc.at[colv.at[j]], add=True)

  plsc.subcore_barrier()
  pltpu.sync_copy(acc.at[pl.ds(s * RPS, RPS)], out_hbm.at[c, pl.ds(s * RPS, RPS)])


_BLK = 1280
_GRID = NP // _BLK


def _mm_body(x_ref, w_ref, o_ref):
  o_ref[...] = jnp.dot(x_ref[...], w_ref[...],
                       preferred_element_type=jnp.float32,
                       precision=lax.Precision.DEFAULT)


def _matmul(x, w):
  return pl.pallas_call(
      _mm_body,
      grid=(_GRID,),
      in_specs=[
          pl.BlockSpec((_BLK, D), lambda i: (i, 0)),
          pl.BlockSpec((D, D), lambda i: (0, 0)),
      ],
      out_specs=pl.BlockSpec((_BLK, D), lambda i: (i, 0)),
      out_shape=jax.ShapeDtypeStruct((NP, D), jnp.float32),
  )(x, w)


def _dis_body(deg_ref, g_ref, dis_ref, hp_ref):
  deg = 1.0 + deg_ref[0] + deg_ref[1]
  dis = jnp.where(deg > 0, lax.rsqrt(deg), 0.0).reshape(_BLK, 1)
  dis_ref[...] = dis
  hp_ref[...] = dis * g_ref[...]


def _dis_scale(deg_part, g):
  return pl.pallas_call(
      _dis_body,
      grid=(_GRID,),
      in_specs=[
          pl.BlockSpec((NC, _BLK), lambda i: (0, i)),
          pl.BlockSpec((_BLK, D), lambda i: (i, 0)),
      ],
      out_specs=[
          pl.BlockSpec((_BLK, 1), lambda i: (i, 0)),
          pl.BlockSpec((_BLK, D), lambda i: (i, 0)),
      ],
      out_shape=[
          jax.ShapeDtypeStruct((NP, 1), jnp.float32),
          jax.ShapeDtypeStruct((NP, D), jnp.float32),
      ],
  )(deg_part, g)


def _epi_body(s_ref, hp_ref, dis_ref, b_ref, w_ref, o_ref):
  dis = dis_ref[...]
  conv = dis * (s_ref[0] + s_ref[1] + hp_ref[...]) + b_ref[...]
  a = jnp.maximum(conv, 0.0)
  g = jnp.dot(a, w_ref[...], preferred_element_type=jnp.float32,
              precision=lax.Precision.DEFAULT)
  o_ref[...] = dis * g


def _epilogue(s_part, hp, dis, b, w_next):
  hn = w_next.shape[1]
  return pl.pallas_call(
      _epi_body,
      grid=(_GRID,),
      in_specs=[
          pl.BlockSpec((NC, _BLK, D), lambda i: (0, i, 0)),
          pl.BlockSpec((_BLK, D), lambda i: (i, 0)),
          pl.BlockSpec((_BLK, 1), lambda i: (i, 0)),
          pl.BlockSpec((1, D), lambda i: (0, 0)),
          pl.BlockSpec((D, hn), lambda i: (0, 0)),
      ],
      out_specs=pl.BlockSpec((_BLK, hn), lambda i: (i, 0)),
      out_shape=jax.ShapeDtypeStruct((NP, hn), jnp.float32),
  )(s_part, hp, dis, b.reshape(1, D), w_next)


def _fin_body(s_ref, hp_ref, dis_ref, b_ref, o_ref):
  s = (s_ref[0] + s_ref[1]).reshape(_BLK, 1)
  o_ref[...] = dis_ref[...] * (s + hp_ref[...]) + b_ref[0, 0]


def _final(s_part, hp, dis, b_fin):
  return pl.pallas_call(
      _fin_body,
      grid=(_GRID,),
      in_specs=[
          pl.BlockSpec((NC, _BLK), lambda i: (0, i)),
          pl.BlockSpec((_BLK, 1), lambda i: (i, 0)),
          pl.BlockSpec((_BLK, 1), lambda i: (i, 0)),
          pl.BlockSpec((1, 1), lambda i: (0, 0)),
      ],
      out_specs=pl.BlockSpec((_BLK, 1), lambda i: (i, 0)),
      out_shape=jax.ShapeDtypeStruct((NP, 1), jnp.float32),
  )(s_part, hp, dis, b_fin.reshape(1, 1))


def kernel(x, edge_index, edge_weight, W_in, b_in, W_mid, b_mid, W_fin, b_fin):
  row = edge_index[0]
  col = edge_index[1]
  pad = EP - E
  row3 = jnp.concatenate([row, jnp.zeros((pad,), jnp.int32)]).reshape(NW, CPW, CH)
  col3 = jnp.concatenate([col, jnp.zeros((pad,), jnp.int32)]).reshape(NW, CPW, CH)
  ew3 = jnp.concatenate(
      [edge_weight, jnp.zeros((pad,), jnp.float32)]).reshape(NW, CPW, CH)
  x_pad = jnp.concatenate([x, jnp.zeros((NP - N, D), jnp.float32)])

  deg_part = _deg_kernel(col3, ew3)
  g1 = _matmul(x_pad, W_in)
  dis, h1p = _dis_scale(deg_part, g1)

  s1 = _agg_kernel(h1p, row3, col3, ew3)
  h2p = _epilogue(s1, h1p, dis, b_in, W_mid)
  s2 = _agg_kernel(h2p, row3, col3, ew3)
  h3p = _epilogue(s2, h2p, dis, b_mid, W_mid)
  s3 = _agg_kernel(h3p, row3, col3, ew3)
  h4p = _epilogue(s3, h3p, dis, b_mid, W_fin)

  s4 = _aggs_kernel(h4p.reshape(NP), row3, col3, ew3)
  out = _final(s4, h4p, dis, b_fin)
  return out[:N]


# speedup vs baseline: 1.0550x; 1.0550x over previous
import dataclasses
import functools

import jax
import jax.numpy as jnp
from jax import lax
from jax.experimental import pallas as pl
from jax.experimental.pallas import tpu as pltpu
from jax.experimental.pallas import tpu_sc as plsc

N = 10000
D = 128
E = 320000

NC = 2
NS = 16
NW = NC * NS
CH = 128
CPW = 80
EP = NW * CPW * CH
NP = 10240
RPS = NP // NS

_mesh = plsc.VectorSubcoreMesh(core_axis_name="c", subcore_axis_name="s")

_cp = pltpu.CompilerParams()
if "needs_layout_passes" in pltpu.CompilerParams.__dataclass_fields__:
  _cp = dataclasses.replace(_cp, needs_layout_passes=False)


@functools.partial(
    pl.kernel,
    out_type=jax.ShapeDtypeStruct((NC, NP), jnp.float32),
    mesh=_mesh,
    scratch_types=[
        pltpu.VMEM((CPW, CH), jnp.int32),
        pltpu.VMEM((CPW, CH), jnp.float32),
        pltpu.VMEM((RPS,), jnp.float32),
        pltpu.VMEM_SHARED((NP,), jnp.float32),
    ],
)
def _deg_kernel(col_hbm, ew_hbm, out_hbm, colv, ewv, zb, acc):
  c = lax.axis_index("c")
  s = lax.axis_index("s")
  w = s * NC + c

  z = jnp.zeros((16,), jnp.float32)

  @pl.loop(0, RPS // 16)
  def _(i):
    zb[pl.ds(i * 16, 16)] = z

  pltpu.sync_copy(zb, acc.at[pl.ds(s * RPS, RPS)])
  plsc.subcore_barrier()

  pltpu.sync_copy(col_hbm.at[w], colv)
  pltpu.sync_copy(ew_hbm.at[w], ewv)

  @pl.loop(0, CPW)
  def _(j):
    pltpu.sync_copy(ewv.at[j], acc.at[colv.at[j]], add=True)

  plsc.subcore_barrier()
  pltpu.sync_copy(acc.at[pl.ds(s * RPS, RPS)], out_hbm.at[c, pl.ds(s * RPS, RPS)])


_PH = 2
_CPP = CPW // _PH
_NSH = 10112
_RSC = _NSH // NS


@functools.partial(
    pl.kernel,
    out_type=jax.ShapeDtypeStruct((NC, NP, D), jnp.float32),
    mesh=_mesh,
    scratch_types=[
        pltpu.VMEM((_CPP, CH), jnp.int32),
        pltpu.VMEM((_CPP, CH), jnp.int32),
        pltpu.VMEM((_CPP, CH), jnp.float32),
        pltpu.VMEM((CH, D), jnp.float32),
        pltpu.VMEM((CH, D), jnp.float32),
        pltpu.VMEM_SHARED((_NSH, D), jnp.float32),
        pltpu.SemaphoreType.DMA,
        pltpu.SemaphoreType.DMA,
    ],
    compiler_params=_cp,
)
def _agg_kernel(hp_hbm, row_hbm, col_hbm, ew_hbm, out_hbm,
                rowv, colv, ewv, gA, gB, acc, semA, semB):
  c = lax.axis_index("c")
  s = lax.axis_index("s")
  w = s * NC + c

  z = jnp.zeros((16,), jnp.float32)

  @pl.loop(0, CH)
  def _(r):
    for v in range(8):
      gA[r, pl.ds(v * 16, 16)] = z

  for t in range(_RSC // CH):
    pltpu.sync_copy(gA, acc.at[pl.ds(s * _RSC + t * CH, CH)])
  if _RSC % CH:
    pltpu.sync_copy(gA.at[pl.ds(0, _RSC % CH)],
                    acc.at[pl.ds(s * _RSC + (_RSC // CH) * CH, _RSC % CH)])
  plsc.subcore_barrier()

  def gstart(j, buf, sem):
    pltpu.make_async_copy(hp_hbm.at[rowv.at[j]], buf, sem).start()

  def gwait(j, buf, sem):
    pltpu.make_async_copy(hp_hbm.at[rowv.at[j]], buf, sem).wait()

  def scale(buf, j):
    @pl.loop(0, CH // 16)
    def _(k):
      for e in range(16):
        idx = k * 16 + e
        spl = plsc.load_gather(
            ewv,
            [jnp.full((16,), j, jnp.int32), jnp.full((16,), idx, jnp.int32)],
        )
        for v in range(8):
          buf[idx, pl.ds(v * 16, 16)] = buf[idx, pl.ds(v * 16, 16)] * spl

  for p in range(_PH):
    pltpu.sync_copy(row_hbm.at[w, pl.ds(p * _CPP, _CPP)], rowv)
    pltpu.sync_copy(col_hbm.at[w, pl.ds(p * _CPP, _CPP)], colv)
    pltpu.sync_copy(ew_hbm.at[w, pl.ds(p * _CPP, _CPP)], ewv)

    gstart(0, gA, semA)
    gstart(1, gB, semB)

    @pl.loop(0, _CPP, step=2)
    def _(j):
      gwait(j, gA, semA)
      scale(gA, j)
      pltpu.sync_copy(gA, acc.at[colv.at[j]], add=True)

      @pl.when(j + 2 < _CPP)
      def _():
        gstart(j + 2, gA, semA)

      gwait(j + 1, gB, semB)
      scale(gB, j + 1)
      pltpu.sync_copy(gB, acc.at[colv.at[j + 1]], add=True)

      @pl.when(j + 3 < _CPP)
      def _():
        gstart(j + 3, gB, semB)

  plsc.subcore_barrier()
  pltpu.sync_copy(acc.at[pl.ds(s * _RSC, _RSC)],
                  out_hbm.at[c, pl.ds(s * _RSC, _RSC)])


@functools.partial(
    pl.kernel,
    out_type=jax.ShapeDtypeStruct((NC, NP), jnp.float32),
    mesh=_mesh,
    scratch_types=[
        pltpu.VMEM((CPW, CH), jnp.int32),
        pltpu.VMEM((CPW, CH), jnp.int32),
        pltpu.VMEM((CPW, CH), jnp.float32),
        pltpu.VMEM((1, CH), jnp.float32),
        pltpu.VMEM((NP,), jnp.float32),
        pltpu.VMEM((RPS,), jnp.float32),
        pltpu.VMEM_SHARED((NP,), jnp.float32),
    ],
    compiler_params=_cp,
)
def _aggs_kernel(h_hbm, row_hbm, col_hbm, ew_hbm, out_hbm,
                 rowv, colv, ewv, msg, tbl, zb, acc):
  c = lax.axis_index("c")
  s = lax.axis_index("s")
  w = s * NC + c

  z = jnp.zeros((16,), jnp.float32)

  @pl.loop(0, RPS // 16)
  def _(i):
    zb[pl.ds(i * 16, 16)] = z

  pltpu.sync_copy(zb, acc.at[pl.ds(s * RPS, RPS)])
  plsc.subcore_barrier()

  pltpu.sync_copy(h_hbm, tbl)
  pltpu.sync_copy(row_hbm.at[w], rowv)
  pltpu.sync_copy(col_hbm.at[w], colv)
  pltpu.sync_copy(ew_hbm.at[w], ewv)

  @pl.loop(0, CPW)
  def _(j):
    @pl.loop(0, CH // 16)
    def _(k):
      row16 = rowv[j, pl.ds(k * 16, 16)]
      ew16 = ewv[j, pl.ds(k * 16, 16)]
      vals = plsc.load_gather(tbl, [row16]) * ew16
      msg[0, pl.ds(k * 16, 16)] = vals

    pltpu.sync_copy(msg.at[0], acc.at[colv.at[j]], add=True)

  plsc.subcore_barrier()
  pltpu.sync_copy(acc.at[pl.ds(s * RPS, RPS)], out_hbm.at[c, pl.ds(s * RPS, RPS)])


_BLK = 1280
_GRID = NP // _BLK


def _mm_body(x_ref, w_ref, o_ref):
  o_ref[...] = jnp.dot(x_ref[...], w_ref[...],
                       preferred_element_type=jnp.float32,
                       precision=lax.Precision.DEFAULT)


def _matmul(x, w):
  return pl.pallas_call(
      _mm_body,
      grid=(_GRID,),
      in_specs=[
          pl.BlockSpec((_BLK, D), lambda i: (i, 0)),
          pl.BlockSpec((D, D), lambda i: (0, 0)),
      ],
      out_specs=pl.BlockSpec((_BLK, D), lambda i: (i, 0)),
      out_shape=jax.ShapeDtypeStruct((NP, D), jnp.float32),
  )(x, w)


def _dis_body(deg_ref, g_ref, dis_ref, hp_ref):
  deg = 1.0 + deg_ref[0] + deg_ref[1]
  dis = jnp.where(deg > 0, lax.rsqrt(deg), 0.0).reshape(_BLK, 1)
  dis_ref[...] = dis
  hp_ref[...] = dis * g_ref[...]


def _dis_scale(deg_part, g):
  return pl.pallas_call(
      _dis_body,
      grid=(_GRID,),
      in_specs=[
          pl.BlockSpec((NC, _BLK), lambda i: (0, i)),
          pl.BlockSpec((_BLK, D), lambda i: (i, 0)),
      ],
      out_specs=[
          pl.BlockSpec((_BLK, 1), lambda i: (i, 0)),
          pl.BlockSpec((_BLK, D), lambda i: (i, 0)),
      ],
      out_shape=[
          jax.ShapeDtypeStruct((NP, 1), jnp.float32),
          jax.ShapeDtypeStruct((NP, D), jnp.float32),
      ],
  )(deg_part, g)


def _epi_body(s_ref, hp_ref, dis_ref, b_ref, w_ref, o_ref):
  dis = dis_ref[...]
  conv = dis * (s_ref[0] + s_ref[1] + hp_ref[...]) + b_ref[...]
  a = jnp.maximum(conv, 0.0)
  g = jnp.dot(a, w_ref[...], preferred_element_type=jnp.float32,
              precision=lax.Precision.DEFAULT)
  o_ref[...] = dis * g


def _epilogue(s_part, hp, dis, b, w_next):
  hn = w_next.shape[1]
  return pl.pallas_call(
      _epi_body,
      grid=(_GRID,),
      in_specs=[
          pl.BlockSpec((NC, _BLK, D), lambda i: (0, i, 0)),
          pl.BlockSpec((_BLK, D), lambda i: (i, 0)),
          pl.BlockSpec((_BLK, 1), lambda i: (i, 0)),
          pl.BlockSpec((1, D), lambda i: (0, 0)),
          pl.BlockSpec((D, hn), lambda i: (0, 0)),
      ],
      out_specs=pl.BlockSpec((_BLK, hn), lambda i: (i, 0)),
      out_shape=jax.ShapeDtypeStruct((NP, hn), jnp.float32),
  )(s_part, hp, dis, b.reshape(1, D), w_next)


def _fin_body(s_ref, hp_ref, dis_ref, b_ref, o_ref):
  s = (s_ref[0] + s_ref[1]).reshape(_BLK, 1)
  o_ref[...] = dis_ref[...] * (s + hp_ref[...]) + b_ref[0, 0]


def _final(s_part, hp, dis, b_fin):
  return pl.pallas_call(
      _fin_body,
      grid=(_GRID,),
      in_specs=[
          pl.BlockSpec((NC, _BLK), lambda i: (0, i)),
          pl.BlockSpec((_BLK, 1), lambda i: (i, 0)),
          pl.BlockSpec((_BLK, 1), lambda i: (i, 0)),
          pl.BlockSpec((1, 1), lambda i: (0, 0)),
      ],
      out_specs=pl.BlockSpec((_BLK, 1), lambda i: (i, 0)),
      out_shape=jax.ShapeDtypeStruct((NP, 1), jnp.float32),
  )(s_part, hp, dis, b_fin.reshape(1, 1))


def kernel(x, edge_index, edge_weight, W_in, b_in, W_mid, b_mid, W_fin, b_fin):
  row = edge_index[0]
  col = edge_index[1]
  pad = EP - E
  row3 = jnp.concatenate([row, jnp.zeros((pad,), jnp.int32)]).reshape(NW, CPW, CH)
  col3 = jnp.concatenate([col, jnp.zeros((pad,), jnp.int32)]).reshape(NW, CPW, CH)
  ew3 = jnp.concatenate(
      [edge_weight, jnp.zeros((pad,), jnp.float32)]).reshape(NW, CPW, CH)
  x_pad = jnp.concatenate([x, jnp.zeros((NP - N, D), jnp.float32)])

  deg_part = _deg_kernel(col3, ew3)
  g1 = _matmul(x_pad, W_in)
  dis, h1p = _dis_scale(deg_part, g1)

  s1 = _agg_kernel(h1p, row3, col3, ew3)
  h2p = _epilogue(s1, h1p, dis, b_in, W_mid)
  s2 = _agg_kernel(h2p, row3, col3, ew3)
  h3p = _epilogue(s2, h2p, dis, b_mid, W_mid)
  s3 = _agg_kernel(h3p, row3, col3, ew3)
  h4p = _epilogue(s3, h3p, dis, b_mid, W_fin)

  s4 = _aggs_kernel(h4p.reshape(NP), row3, col3, ew3)
  out = _final(s4, h4p, dis, b_fin)
  return out[:N]
